# baseline (speedup 1.0000x reference)
import functools

import jax
import jax.numpy as jnp
from jax import lax
from jax.experimental import pallas as pl
from jax.experimental.pallas import tpu as pltpu
from jax.experimental.pallas import tpu_sc as plsc

f32 = jnp.float32

N = 10000
NPAD = 10240
RPT = NPAD // 16
ALPHA = 0.2
NFEAT = 128
F1 = 256
FH = 128
NCLASS = 40
FP = 128
E = 320000
CW = 128
CH1 = 160
IB1 = 16
CHD = 80
CH2 = 80
IB2 = 16
NB = 2
E1 = 16 * CH1 * CW
E2 = 32 * CH2 * CW
BLK = 512

_mesh = plsc.VectorSubcoreMesh(core_axis_name="c", subcore_axis_name="s")


def _fill(ref, rows, cols, value):
    @pl.loop(0, rows)
    def _(r):
        @pl.loop(0, cols // 16)
        def _(q):
            ref[r, pl.ds(q * 16, 16)] = jnp.full((16,), value, f32)


def _zero_acc_slice(zbuf, acc, base):
    @pl.loop(0, RPT // 128)
    def _(k):
        pltpu.sync_copy(zbuf, acc.at[pl.ds(base + k * 128, 128)])


def _zero_acc2(zb0, acc, base):
    @pl.loop(0, RPT // CW)
    def _(k):
        pltpu.sync_copy(zb0, acc.at[pl.ds(base + k * CW, CW)])


@functools.partial(
    pl.kernel,
    out_type=(jax.ShapeDtypeStruct((NPAD, 16), f32),
              jax.ShapeDtypeStruct((NPAD, 16), f32)),
    mesh=_mesh,
    scratch_types=[pltpu.VMEM((CHD, 128), jnp.int32),
                   pltpu.VMEM((128, 16), f32),
                   pltpu.VMEM_SHARED((NPAD, 16), f32)],
)
def _deg_kernel(dst_hbm, o0_hbm, o1_hbm, idx_v, buf_v, acc_s):
    c = lax.axis_index("c")
    s = lax.axis_index("s")
    wid = c * 16 + s
    base = s * RPT

    _fill(buf_v, 128, 16, 0.0)
    _zero_acc_slice(buf_v, acc_s, base)
    pltpu.sync_copy(dst_hbm.at[wid], idx_v)
    _fill(buf_v, 128, 16, 1.0)
    plsc.subcore_barrier()

    @pl.loop(0, CHD)
    def _(j):
        pltpu.sync_copy(buf_v, acc_s.at[idx_v.at[j]], add=True)

    plsc.subcore_barrier()

    @pl.when(c == 0)
    def _():
        pltpu.sync_copy(acc_s.at[pl.ds(base, RPT)], o0_hbm.at[pl.ds(base, RPT)])

    @pl.when(c == 1)
    def _():
        pltpu.sync_copy(acc_s.at[pl.ds(base, RPT)], o1_hbm.at[pl.ds(base, RPT)])


@functools.partial(
    pl.kernel,
    out_type=(jax.ShapeDtypeStruct((NPAD, FH), f32),
              jax.ShapeDtypeStruct((NPAD, FH), f32)),
    mesh=_mesh,
    scratch_types=[pltpu.VMEM((IB1, CW), jnp.int32),
                   pltpu.VMEM((IB1, CW), jnp.int32)]
                  + [pltpu.VMEM((CW, FH), f32)] * NB
                  + [pltpu.VMEM_SHARED((NPAD, FH), f32)]
                  + [pltpu.SemaphoreType.DMA] * (2 * NB),
)
def _agg1_kernel(hsa_hbm, hsb_hbm, src_hbm, dst_hbm, oa_hbm, ob_hbm,
                 src_v, dst_v, *rest):
    gbufs = rest[:NB]
    acc_s = rest[NB]
    gsems = rest[NB + 1:2 * NB + 1]
    ssems = rest[2 * NB + 1:]
    c = lax.axis_index("c")
    s = lax.axis_index("s")
    base = s * RPT

    _fill(gbufs[0], CW, FH, 0.0)
    _zero_acc2(gbufs[0], acc_s, base)
    plsc.subcore_barrier()

    def edge_loop(tbl):
        def swait(b):
            pltpu.make_async_copy(tbl.at[pl.ds(0, CW)],
                                  acc_s.at[pl.ds(0, CW)], ssems[b]).wait()

        @pl.loop(0, CH1 // IB1)
        def _(r):
            pltpu.sync_copy(src_hbm.at[s, pl.ds(r * IB1, IB1)], src_v)
            pltpu.sync_copy(dst_hbm.at[s, pl.ds(r * IB1, IB1)], dst_v)
            for b in range(NB):
                @pl.when(r > 0)
                def _():
                    swait(b)
                pltpu.async_copy(tbl.at[src_v.at[b]], gbufs[b], gsems[b])

            @pl.loop(0, IB1 // NB)
            def _(t):
                for b in range(NB):
                    j = NB * t + b
                    pltpu.make_async_copy(tbl.at[pl.ds(0, CW)], gbufs[b],
                                          gsems[b]).wait()
                    pltpu.async_copy(gbufs[b], acc_s.at[dst_v.at[j]],
                                     ssems[b], add=True)
                    bp = (b + NB - 1) % NB
                    k = j + NB - 1

                    @pl.when((j >= 1) & (k < IB1))
                    def _():
                        swait(bp)
                        pltpu.async_copy(tbl.at[src_v.at[k]], gbufs[bp],
                                         gsems[bp])

        for b in range(NB):
            swait(b)

    @pl.when(c == 0)
    def _():
        edge_loop(hsa_hbm)

    @pl.when(c == 1)
    def _():
        edge_loop(hsb_hbm)

    plsc.subcore_barrier()

    @pl.when(c == 0)
    def _():
        pltpu.sync_copy(acc_s.at[pl.ds(base, RPT)], oa_hbm.at[pl.ds(base, RPT)])

    @pl.when(c == 1)
    def _():
        pltpu.sync_copy(acc_s.at[pl.ds(base, RPT)], ob_hbm.at[pl.ds(base, RPT)])


@functools.partial(
    pl.kernel,
    out_type=(jax.ShapeDtypeStruct((NPAD, FP), f32),
              jax.ShapeDtypeStruct((NPAD, FP), f32)),
    mesh=_mesh,
    scratch_types=[pltpu.VMEM((IB2, CW), jnp.int32),
                   pltpu.VMEM((IB2, CW), jnp.int32)]
                  + [pltpu.VMEM((CW, FP), f32)] * NB
                  + [pltpu.VMEM_SHARED((NPAD, FP), f32)]
                  + [pltpu.SemaphoreType.DMA] * (2 * NB),
)
def _agg2_kernel(hs2_hbm, src_hbm, dst_hbm, o0_hbm, o1_hbm,
                 src_v, dst_v, *rest):
    gbufs = rest[:NB]
    acc_s = rest[NB]
    gsems = rest[NB + 1:2 * NB + 1]
    ssems = rest[2 * NB + 1:]
    c = lax.axis_index("c")
    s = lax.axis_index("s")
    wid = c * 16 + s
    base = s * RPT

    _fill(gbufs[0], CW, FP, 0.0)
    _zero_acc2(gbufs[0], acc_s, base)
    plsc.subcore_barrier()

    def swait(b):
        pltpu.make_async_copy(hs2_hbm.at[pl.ds(0, CW)],
                              acc_s.at[pl.ds(0, CW)], ssems[b]).wait()

    @pl.loop(0, CH2 // IB2)
    def _(r):
        pltpu.sync_copy(src_hbm.at[wid, pl.ds(r * IB2, IB2)], src_v)
        pltpu.sync_copy(dst_hbm.at[wid, pl.ds(r * IB2, IB2)], dst_v)
        for b in range(NB):
            @pl.when(r > 0)
            def _():
                swait(b)
            pltpu.async_copy(hs2_hbm.at[src_v.at[b]], gbufs[b], gsems[b])

        @pl.loop(0, IB2 // NB)
        def _(t):
            for b in range(NB):
                j = NB * t + b
                pltpu.make_async_copy(hs2_hbm.at[pl.ds(0, CW)], gbufs[b],
                                      gsems[b]).wait()
                pltpu.async_copy(gbufs[b], acc_s.at[dst_v.at[j]],
                                 ssems[b], add=True)
                bp = (b + NB - 1) % NB
                k = j + NB - 1

                @pl.when((j >= 1) & (k < IB2))
                def _():
                    swait(bp)
                    pltpu.async_copy(hs2_hbm.at[src_v.at[k]], gbufs[bp],
                                     gsems[bp])

    for b in range(NB):
        swait(b)

    plsc.subcore_barrier()

    @pl.when(c == 0)
    def _():
        pltpu.sync_copy(acc_s.at[pl.ds(base, RPT)], o0_hbm.at[pl.ds(base, RPT)])

    @pl.when(c == 1)
    def _():
        pltpu.sync_copy(acc_s.at[pl.ds(base, RPT)], o1_hbm.at[pl.ds(base, RPT)])


def _dinv(d0_ref, d1_ref):
    deg = d0_ref[:, 0:1] + d1_ref[:, 0:1] + 1.0
    return lax.rsqrt(deg)


def _dense1_body(x_ref, wl_ref, bl_ref, w1_ref, o_ref):
    h0 = jnp.dot(x_ref[...], wl_ref[...], preferred_element_type=f32) + bl_ref[...]
    h0 = jnp.maximum(h0, ALPHA * h0)
    o_ref[...] = jnp.dot(h0, w1_ref[...], preferred_element_type=f32)


def _scale_body(h2_ref, d0_ref, d1_ref, oa_ref, ob_ref):
    hs = h2_ref[...] * _dinv(d0_ref, d1_ref)
    oa_ref[...] = hs[:, :FH]
    ob_ref[...] = hs[:, FH:]


def _dense2_body(ga_ref, gb_ref, ha_ref, hb_ref, d0_ref, d1_ref, b1_ref,
                 w2_ref, o_ref):
    dinv = _dinv(d0_ref, d1_ref)
    t = jnp.concatenate([ga_ref[...] + ha_ref[...], gb_ref[...] + hb_ref[...]],
                        axis=1)
    t = t * dinv + b1_ref[...]
    h1 = jnp.maximum(t, ALPHA * t)
    o_ref[...] = jnp.dot(h1, w2_ref[...], preferred_element_type=f32) * dinv


def _final_body(p0_ref, p1_ref, hs2_ref, d0_ref, d1_ref, b2_ref, o_ref):
    z = (p0_ref[...] + p1_ref[...] + hs2_ref[...]) * _dinv(d0_ref, d1_ref)
    z = z + b2_ref[...]
    col = lax.broadcasted_iota(jnp.int32, z.shape, 1)
    valid = col < NCLASS
    zm = jnp.where(valid, z, -jnp.inf)
    m = jnp.max(zm, axis=1, keepdims=True)
    e = jnp.where(valid, jnp.exp(z - m), 0.0)
    lse = jnp.log(jnp.sum(e, axis=1, keepdims=True)) + m
    o_ref[...] = z - lse


def _row_spec(width):
    return pl.BlockSpec((BLK, width), lambda i: (i, 0))


def _full_spec(shape):
    return pl.BlockSpec(shape, lambda i: (0, 0))


_GRID = (NPAD // BLK,)


def _dense1(x_p, Wl, bl2, W1):
    return pl.pallas_call(
        _dense1_body,
        grid=_GRID,
        in_specs=[_row_spec(NFEAT), _full_spec((NFEAT, F1)),
                  _full_spec((1, F1)), _full_spec((F1, F1))],
        out_specs=_row_spec(F1),
        out_shape=jax.ShapeDtypeStruct((NPAD, F1), f32),
    )(x_p, Wl, bl2, W1)


def _scale_split(h2, d0, d1):
    return pl.pallas_call(
        _scale_body,
        grid=_GRID,
        in_specs=[_row_spec(F1), _row_spec(16), _row_spec(16)],
        out_specs=[_row_spec(FH)] * 2,
        out_shape=[jax.ShapeDtypeStruct((NPAD, FH), f32)] * 2,
    )(h2, d0, d1)


def _dense2(ga, gb, ha, hb, d0, d1, b12, W2p):
    return pl.pallas_call(
        _dense2_body,
        grid=_GRID,
        in_specs=[_row_spec(FH), _row_spec(FH), _row_spec(FH), _row_spec(FH),
                  _row_spec(16), _row_spec(16), _full_spec((1, F1)),
                  _full_spec((F1, FP))],
        out_specs=_row_spec(FP),
        out_shape=jax.ShapeDtypeStruct((NPAD, FP), f32),
    )(ga, gb, ha, hb, d0, d1, b12, W2p)


def _final(p0, p1, hs2, d0, d1, b22):
    return pl.pallas_call(
        _final_body,
        grid=_GRID,
        in_specs=[_row_spec(FP), _row_spec(FP), _row_spec(FP),
                  _row_spec(16), _row_spec(16), _full_spec((1, FP))],
        out_specs=_row_spec(FP),
        out_shape=jax.ShapeDtypeStruct((NPAD, FP), f32),
    )(p0, p1, hs2, d0, d1, b22)


def kernel(x, adj, edge_index, Wl, bl, W1, b1, W2, b2):
    del adj
    x_p = jnp.pad(x, ((0, NPAD - N), (0, 0)))
    src = edge_index[0]
    dst = edge_index[1]
    fill1 = jnp.full((E1 - E,), N, jnp.int32)
    fill2 = jnp.full((E2 - E,), N, jnp.int32)
    src1 = jnp.concatenate([src, fill1]).reshape(16, CH1, CW)
    dst1 = jnp.concatenate([dst, fill1]).reshape(16, CH1, CW)
    src2 = jnp.concatenate([src, fill2]).reshape(32, CH2, CW)
    dst2 = jnp.concatenate([dst, fill2]).reshape(32, CH2, CW)
    dstd = jnp.concatenate([dst, fill2]).reshape(32, CHD, 128)
    bl2 = bl.reshape(1, F1)
    b12 = b1.reshape(1, F1)
    b22 = jnp.pad(b2, (0, FP - NCLASS)).reshape(1, FP)
    W2p = jnp.pad(W2, ((0, 0), (0, FP - NCLASS)))

    d0, d1 = _deg_kernel(dstd)
    h2 = _dense1(x_p, Wl, bl2, W1)
    hsa, hsb = _scale_split(h2, d0, d1)
    ga, gb = _agg1_kernel(hsa, hsb, src1, dst1)
    hs2 = _dense2(ga, gb, hsa, hsb, d0, d1, b12, W2p)
    p0, p1 = _agg2_kernel(hs2, src2, dst2)
    out = _final(p0, p1, hs2, d0, d1, b22)
    return out[:N, :NCLASS]


# speedup vs baseline: 1.0194x; 1.0194x over previous
import functools

import jax
import jax.numpy as jnp
from jax import lax
from jax.experimental import pallas as pl
from jax.experimental.pallas import tpu as pltpu
from jax.experimental.pallas import tpu_sc as plsc

f32 = jnp.float32

N = 10000
NPAD = 10240
RPT = NPAD // 16
ALPHA = 0.2
NFEAT = 128
F1 = 256
FH = 128
NCLASS = 40
FP = 128
E = 320000
CW = 32
CH1 = 640
IB1 = 64
CHD = 80
CH2 = 320
IB2 = 64
NB = 8
E1 = 16 * CH1 * CW
E2 = 32 * CH2 * CW
BLK = 512

_mesh = plsc.VectorSubcoreMesh(core_axis_name="c", subcore_axis_name="s")


def _fill(ref, rows, cols, value):
    @pl.loop(0, rows)
    def _(r):
        @pl.loop(0, cols // 16)
        def _(q):
            ref[r, pl.ds(q * 16, 16)] = jnp.full((16,), value, f32)


def _zero_acc_slice(zbuf, acc, base):
    @pl.loop(0, RPT // 128)
    def _(k):
        pltpu.sync_copy(zbuf, acc.at[pl.ds(base + k * 128, 128)])


def _zero_acc2(zb0, acc, base):
    @pl.loop(0, RPT // CW)
    def _(k):
        pltpu.sync_copy(zb0, acc.at[pl.ds(base + k * CW, CW)])


@functools.partial(
    pl.kernel,
    out_type=(jax.ShapeDtypeStruct((NPAD, 16), f32),
              jax.ShapeDtypeStruct((NPAD, 16), f32)),
    mesh=_mesh,
    scratch_types=[pltpu.VMEM((CHD, 128), jnp.int32),
                   pltpu.VMEM((128, 16), f32),
                   pltpu.VMEM_SHARED((NPAD, 16), f32)],
)
def _deg_kernel(dst_hbm, o0_hbm, o1_hbm, idx_v, buf_v, acc_s):
    c = lax.axis_index("c")
    s = lax.axis_index("s")
    wid = c * 16 + s
    base = s * RPT

    _fill(buf_v, 128, 16, 0.0)
    _zero_acc_slice(buf_v, acc_s, base)
    pltpu.sync_copy(dst_hbm.at[wid], idx_v)
    _fill(buf_v, 128, 16, 1.0)
    plsc.subcore_barrier()

    @pl.loop(0, CHD)
    def _(j):
        pltpu.sync_copy(buf_v, acc_s.at[idx_v.at[j]], add=True)

    plsc.subcore_barrier()

    @pl.when(c == 0)
    def _():
        pltpu.sync_copy(acc_s.at[pl.ds(base, RPT)], o0_hbm.at[pl.ds(base, RPT)])

    @pl.when(c == 1)
    def _():
        pltpu.sync_copy(acc_s.at[pl.ds(base, RPT)], o1_hbm.at[pl.ds(base, RPT)])


@functools.partial(
    pl.kernel,
    out_type=(jax.ShapeDtypeStruct((NPAD, FH), f32),
              jax.ShapeDtypeStruct((NPAD, FH), f32)),
    mesh=_mesh,
    scratch_types=[pltpu.VMEM((IB1, CW), jnp.int32),
                   pltpu.VMEM((IB1, CW), jnp.int32)]
                  + [pltpu.VMEM((CW, FH), f32)] * NB
                  + [pltpu.VMEM_SHARED((NPAD, FH), f32)]
                  + [pltpu.SemaphoreType.DMA] * (2 * NB),
)
def _agg1_kernel(hsa_hbm, hsb_hbm, src_hbm, dst_hbm, oa_hbm, ob_hbm,
                 src_v, dst_v, *rest):
    gbufs = rest[:NB]
    acc_s = rest[NB]
    gsems = rest[NB + 1:2 * NB + 1]
    ssems = rest[2 * NB + 1:]
    c = lax.axis_index("c")
    s = lax.axis_index("s")
    base = s * RPT

    _fill(gbufs[0], CW, FH, 0.0)
    _zero_acc2(gbufs[0], acc_s, base)
    plsc.subcore_barrier()

    def edge_loop(tbl):
        def swait(b):
            pltpu.make_async_copy(tbl.at[pl.ds(0, CW)],
                                  acc_s.at[pl.ds(0, CW)], ssems[b]).wait()

        @pl.loop(0, CH1 // IB1)
        def _(r):
            pltpu.sync_copy(src_hbm.at[s, pl.ds(r * IB1, IB1)], src_v)
            pltpu.sync_copy(dst_hbm.at[s, pl.ds(r * IB1, IB1)], dst_v)
            for b in range(NB):
                @pl.when(r > 0)
                def _():
                    swait(b)
                pltpu.async_copy(tbl.at[src_v.at[b]], gbufs[b], gsems[b])

            @pl.loop(0, IB1 // NB)
            def _(t):
                for b in range(NB):
                    j = NB * t + b
                    pltpu.make_async_copy(tbl.at[pl.ds(0, CW)], gbufs[b],
                                          gsems[b]).wait()
                    pltpu.async_copy(gbufs[b], acc_s.at[dst_v.at[j]],
                                     ssems[b], add=True)
                    bp = (b + NB - 1) % NB
                    k = j + NB - 1

                    @pl.when((j >= 1) & (k < IB1))
                    def _():
                        swait(bp)
                        pltpu.async_copy(tbl.at[src_v.at[k]], gbufs[bp],
                                         gsems[bp])

        for b in range(NB):
            swait(b)

    @pl.when(c == 0)
    def _():
        edge_loop(hsa_hbm)

    @pl.when(c == 1)
    def _():
        edge_loop(hsb_hbm)

    plsc.subcore_barrier()

    @pl.when(c == 0)
    def _():
        pltpu.sync_copy(acc_s.at[pl.ds(base, RPT)], oa_hbm.at[pl.ds(base, RPT)])

    @pl.when(c == 1)
    def _():
        pltpu.sync_copy(acc_s.at[pl.ds(base, RPT)], ob_hbm.at[pl.ds(base, RPT)])


@functools.partial(
    pl.kernel,
    out_type=(jax.ShapeDtypeStruct((NPAD, FP), f32),
              jax.ShapeDtypeStruct((NPAD, FP), f32)),
    mesh=_mesh,
    scratch_types=[pltpu.VMEM((IB2, CW), jnp.int32),
                   pltpu.VMEM((IB2, CW), jnp.int32)]
                  + [pltpu.VMEM((CW, FP), f32)] * NB
                  + [pltpu.VMEM_SHARED((NPAD, FP), f32)]
                  + [pltpu.SemaphoreType.DMA] * (2 * NB),
)
def _agg2_kernel(hs2_hbm, src_hbm, dst_hbm, o0_hbm, o1_hbm,
                 src_v, dst_v, *rest):
    gbufs = rest[:NB]
    acc_s = rest[NB]
    gsems = rest[NB + 1:2 * NB + 1]
    ssems = rest[2 * NB + 1:]
    c = lax.axis_index("c")
    s = lax.axis_index("s")
    wid = c * 16 + s
    base = s * RPT

    _fill(gbufs[0], CW, FP, 0.0)
    _zero_acc2(gbufs[0], acc_s, base)
    plsc.subcore_barrier()

    def swait(b):
        pltpu.make_async_copy(hs2_hbm.at[pl.ds(0, CW)],
                              acc_s.at[pl.ds(0, CW)], ssems[b]).wait()

    @pl.loop(0, CH2 // IB2)
    def _(r):
        pltpu.sync_copy(src_hbm.at[wid, pl.ds(r * IB2, IB2)], src_v)
        pltpu.sync_copy(dst_hbm.at[wid, pl.ds(r * IB2, IB2)], dst_v)
        for b in range(NB):
            @pl.when(r > 0)
            def _():
                swait(b)
            pltpu.async_copy(hs2_hbm.at[src_v.at[b]], gbufs[b], gsems[b])

        @pl.loop(0, IB2 // NB)
        def _(t):
            for b in range(NB):
                j = NB * t + b
                pltpu.make_async_copy(hs2_hbm.at[pl.ds(0, CW)], gbufs[b],
                                      gsems[b]).wait()
                pltpu.async_copy(gbufs[b], acc_s.at[dst_v.at[j]],
                                 ssems[b], add=True)
                bp = (b + NB - 1) % NB
                k = j + NB - 1

                @pl.when((j >= 1) & (k < IB2))
                def _():
                    swait(bp)
                    pltpu.async_copy(hs2_hbm.at[src_v.at[k]], gbufs[bp],
                                     gsems[bp])

    for b in range(NB):
        swait(b)

    plsc.subcore_barrier()

    @pl.when(c == 0)
    def _():
        pltpu.sync_copy(acc_s.at[pl.ds(base, RPT)], o0_hbm.at[pl.ds(base, RPT)])

    @pl.when(c == 1)
    def _():
        pltpu.sync_copy(acc_s.at[pl.ds(base, RPT)], o1_hbm.at[pl.ds(base, RPT)])


def _dinv(d0_ref, d1_ref):
    deg = d0_ref[:, 0:1] + d1_ref[:, 0:1] + 1.0
    return lax.rsqrt(deg)


def _dense1_body(x_ref, wl_ref, bl_ref, w1_ref, o_ref):
    h0 = jnp.dot(x_ref[...], wl_ref[...], preferred_element_type=f32) + bl_ref[...]
    h0 = jnp.maximum(h0, ALPHA * h0)
    o_ref[...] = jnp.dot(h0, w1_ref[...], preferred_element_type=f32)


def _scale_body(h2_ref, d0_ref, d1_ref, oa_ref, ob_ref):
    hs = h2_ref[...] * _dinv(d0_ref, d1_ref)
    oa_ref[...] = hs[:, :FH]
    ob_ref[...] = hs[:, FH:]


def _dense2_body(ga_ref, gb_ref, ha_ref, hb_ref, d0_ref, d1_ref, b1_ref,
                 w2_ref, o_ref):
    dinv = _dinv(d0_ref, d1_ref)
    t = jnp.concatenate([ga_ref[...] + ha_ref[...], gb_ref[...] + hb_ref[...]],
                        axis=1)
    t = t * dinv + b1_ref[...]
    h1 = jnp.maximum(t, ALPHA * t)
    o_ref[...] = jnp.dot(h1, w2_ref[...], preferred_element_type=f32) * dinv


def _final_body(p0_ref, p1_ref, hs2_ref, d0_ref, d1_ref, b2_ref, o_ref):
    z = (p0_ref[...] + p1_ref[...] + hs2_ref[...]) * _dinv(d0_ref, d1_ref)
    z = z + b2_ref[...]
    col = lax.broadcasted_iota(jnp.int32, z.shape, 1)
    valid = col < NCLASS
    zm = jnp.where(valid, z, -jnp.inf)
    m = jnp.max(zm, axis=1, keepdims=True)
    e = jnp.where(valid, jnp.exp(z - m), 0.0)
    lse = jnp.log(jnp.sum(e, axis=1, keepdims=True)) + m
    o_ref[...] = z - lse


def _row_spec(width):
    return pl.BlockSpec((BLK, width), lambda i: (i, 0))


def _full_spec(shape):
    return pl.BlockSpec(shape, lambda i: (0, 0))


_GRID = (NPAD // BLK,)


def _dense1(x_p, Wl, bl2, W1):
    return pl.pallas_call(
        _dense1_body,
        grid=_GRID,
        in_specs=[_row_spec(NFEAT), _full_spec((NFEAT, F1)),
                  _full_spec((1, F1)), _full_spec((F1, F1))],
        out_specs=_row_spec(F1),
        out_shape=jax.ShapeDtypeStruct((NPAD, F1), f32),
    )(x_p, Wl, bl2, W1)


def _scale_split(h2, d0, d1):
    return pl.pallas_call(
        _scale_body,
        grid=_GRID,
        in_specs=[_row_spec(F1), _row_spec(16), _row_spec(16)],
        out_specs=[_row_spec(FH)] * 2,
        out_shape=[jax.ShapeDtypeStruct((NPAD, FH), f32)] * 2,
    )(h2, d0, d1)


def _dense2(ga, gb, ha, hb, d0, d1, b12, W2p):
    return pl.pallas_call(
        _dense2_body,
        grid=_GRID,
        in_specs=[_row_spec(FH), _row_spec(FH), _row_spec(FH), _row_spec(FH),
                  _row_spec(16), _row_spec(16), _full_spec((1, F1)),
                  _full_spec((F1, FP))],
        out_specs=_row_spec(FP),
        out_shape=jax.ShapeDtypeStruct((NPAD, FP), f32),
    )(ga, gb, ha, hb, d0, d1, b12, W2p)


def _final(p0, p1, hs2, d0, d1, b22):
    return pl.pallas_call(
        _final_body,
        grid=_GRID,
        in_specs=[_row_spec(FP), _row_spec(FP), _row_spec(FP),
                  _row_spec(16), _row_spec(16), _full_spec((1, FP))],
        out_specs=_row_spec(FP),
        out_shape=jax.ShapeDtypeStruct((NPAD, FP), f32),
    )(p0, p1, hs2, d0, d1, b22)


def kernel(x, adj, edge_index, Wl, bl, W1, b1, W2, b2):
    del adj
    x_p = jnp.pad(x, ((0, NPAD - N), (0, 0)))
    src = edge_index[0]
    dst = edge_index[1]
    fill1 = jnp.full((E1 - E,), N, jnp.int32)
    fill2 = jnp.full((E2 - E,), N, jnp.int32)
    src1 = jnp.concatenate([src, fill1]).reshape(16, CH1, CW)
    dst1 = jnp.concatenate([dst, fill1]).reshape(16, CH1, CW)
    src2 = jnp.concatenate([src, fill2]).reshape(32, CH2, CW)
    dst2 = jnp.concatenate([dst, fill2]).reshape(32, CH2, CW)
    dstd = jnp.concatenate([dst, fill2]).reshape(32, CHD, 128)
    bl2 = bl.reshape(1, F1)
    b12 = b1.reshape(1, F1)
    b22 = jnp.pad(b2, (0, FP - NCLASS)).reshape(1, FP)
    W2p = jnp.pad(W2, ((0, 0), (0, FP - NCLASS)))

    d0, d1 = _deg_kernel(dstd)
    h2 = _dense1(x_p, Wl, bl2, W1)
    hsa, hsb = _scale_split(h2, d0, d1)
    ga, gb = _agg1_kernel(hsa, hsb, src1, dst1)
    hs2 = _dense2(ga, gb, hsa, hsb, d0, d1, b12, W2p)
    p0, p1 = _agg2_kernel(hs2, src2, dst2)
    out = _final(p0, p1, hs2, d0, d1, b22)
    return out[:N, :NCLASS]


# speedup vs baseline: 1.1266x; 1.1052x over previous
import functools

import jax
import jax.numpy as jnp
from jax import lax
from jax.experimental import pallas as pl
from jax.experimental.pallas import tpu as pltpu
from jax.experimental.pallas import tpu_sc as plsc

f32 = jnp.float32

N = 10000
NPAD = 10240
RPT = NPAD // 16
ALPHA = 0.2
NFEAT = 128
F1 = 256
FH = 128
NCLASS = 40
FP = 128
E = 320000
CW = 64
CH1 = 320
IB1 = 32
CHD = 80
CH2 = 160
IB2 = 32
NB = 4
E1 = 16 * CH1 * CW
E2 = 32 * CH2 * CW
BLK = 512

_mesh = plsc.VectorSubcoreMesh(core_axis_name="c", subcore_axis_name="s")


def _fill(ref, rows, cols, value):
    @pl.loop(0, rows)
    def _(r):
        @pl.loop(0, cols // 16)
        def _(q):
            ref[r, pl.ds(q * 16, 16)] = jnp.full((16,), value, f32)


def _zero_acc_slice(zbuf, acc, base):
    @pl.loop(0, RPT // 128)
    def _(k):
        pltpu.sync_copy(zbuf, acc.at[pl.ds(base + k * 128, 128)])


def _zero_acc2(zb0, acc, base):
    @pl.loop(0, RPT // CW)
    def _(k):
        pltpu.sync_copy(zb0, acc.at[pl.ds(base + k * CW, CW)])


@functools.partial(
    pl.kernel,
    out_type=(jax.ShapeDtypeStruct((NPAD, 16), f32),
              jax.ShapeDtypeStruct((NPAD, 16), f32)),
    mesh=_mesh,
    scratch_types=[pltpu.VMEM((CHD, 128), jnp.int32),
                   pltpu.VMEM((128, 16), f32),
                   pltpu.VMEM_SHARED((NPAD, 16), f32)],
)
def _deg_kernel(dst_hbm, o0_hbm, o1_hbm, idx_v, buf_v, acc_s):
    c = lax.axis_index("c")
    s = lax.axis_index("s")
    wid = c * 16 + s
    base = s * RPT

    _fill(buf_v, 128, 16, 0.0)
    _zero_acc_slice(buf_v, acc_s, base)
    pltpu.sync_copy(dst_hbm.at[wid], idx_v)
    _fill(buf_v, 128, 16, 1.0)
    plsc.subcore_barrier()

    @pl.loop(0, CHD)
    def _(j):
        pltpu.sync_copy(buf_v, acc_s.at[idx_v.at[j]], add=True)

    plsc.subcore_barrier()

    @pl.when(c == 0)
    def _():
        pltpu.sync_copy(acc_s.at[pl.ds(base, RPT)], o0_hbm.at[pl.ds(base, RPT)])

    @pl.when(c == 1)
    def _():
        pltpu.sync_copy(acc_s.at[pl.ds(base, RPT)], o1_hbm.at[pl.ds(base, RPT)])


@functools.partial(
    pl.kernel,
    out_type=(jax.ShapeDtypeStruct((NPAD, FH), f32),
              jax.ShapeDtypeStruct((NPAD, FH), f32)),
    mesh=_mesh,
    scratch_types=[pltpu.VMEM((IB1, CW), jnp.int32),
                   pltpu.VMEM((IB1, CW), jnp.int32)]
                  + [pltpu.VMEM((CW, FH), f32)] * NB
                  + [pltpu.VMEM_SHARED((NPAD, FH), f32)]
                  + [pltpu.SemaphoreType.DMA] * (2 * NB),
)
def _agg1_kernel(hsa_hbm, hsb_hbm, src_hbm, dst_hbm, oa_hbm, ob_hbm,
                 src_v, dst_v, *rest):
    gbufs = rest[:NB]
    acc_s = rest[NB]
    gsems = rest[NB + 1:2 * NB + 1]
    ssems = rest[2 * NB + 1:]
    c = lax.axis_index("c")
    s = lax.axis_index("s")
    base = s * RPT

    _fill(gbufs[0], CW, FH, 0.0)
    _zero_acc2(gbufs[0], acc_s, base)
    plsc.subcore_barrier()

    def edge_loop(tbl):
        def swait(b):
            pltpu.make_async_copy(tbl.at[pl.ds(0, CW)],
                                  acc_s.at[pl.ds(0, CW)], ssems[b]).wait()

        @pl.loop(0, CH1 // IB1)
        def _(r):
            pltpu.sync_copy(src_hbm.at[s, pl.ds(r * IB1, IB1)], src_v)
            pltpu.sync_copy(dst_hbm.at[s, pl.ds(r * IB1, IB1)], dst_v)
            for b in range(NB):
                @pl.when(r > 0)
                def _():
                    swait(b)
                pltpu.async_copy(tbl.at[src_v.at[b]], gbufs[b], gsems[b])

            @pl.loop(0, IB1 // NB)
            def _(t):
                for b in range(NB):
                    j = NB * t + b
                    pltpu.make_async_copy(tbl.at[pl.ds(0, CW)], gbufs[b],
                                          gsems[b]).wait()
                    pltpu.async_copy(gbufs[b], acc_s.at[dst_v.at[j]],
                                     ssems[b], add=True)
                    bp = (b + NB - 1) % NB
                    k = j + NB - 1

                    @pl.when((j >= 1) & (k < IB1))
                    def _():
                        swait(bp)
                        pltpu.async_copy(tbl.at[src_v.at[k]], gbufs[bp],
                                         gsems[bp])

        for b in range(NB):
            swait(b)

    @pl.when(c == 0)
    def _():
        edge_loop(hsa_hbm)

    @pl.when(c == 1)
    def _():
        edge_loop(hsb_hbm)

    plsc.subcore_barrier()

    @pl.when(c == 0)
    def _():
        pltpu.sync_copy(acc_s.at[pl.ds(base, RPT)], oa_hbm.at[pl.ds(base, RPT)])

    @pl.when(c == 1)
    def _():
        pltpu.sync_copy(acc_s.at[pl.ds(base, RPT)], ob_hbm.at[pl.ds(base, RPT)])


@functools.partial(
    pl.kernel,
    out_type=(jax.ShapeDtypeStruct((NPAD, FP), f32),
              jax.ShapeDtypeStruct((NPAD, FP), f32)),
    mesh=_mesh,
    scratch_types=[pltpu.VMEM((IB2, CW), jnp.int32),
                   pltpu.VMEM((IB2, CW), jnp.int32)]
                  + [pltpu.VMEM((CW, FP), f32)] * NB
                  + [pltpu.VMEM_SHARED((NPAD, FP), f32)]
                  + [pltpu.SemaphoreType.DMA] * (2 * NB),
)
def _agg2_kernel(hs2a_hbm, hs2b_hbm, src_hbm, dst_hbm, o0_hbm, o1_hbm,
                 src_v, dst_v, *rest):
    gbufs = rest[:NB]
    acc_s = rest[NB]
    gsems = rest[NB + 1:2 * NB + 1]
    ssems = rest[2 * NB + 1:]
    c = lax.axis_index("c")
    s = lax.axis_index("s")
    wid = c * 16 + s
    base = s * RPT

    _fill(gbufs[0], CW, FP, 0.0)
    _zero_acc2(gbufs[0], acc_s, base)
    plsc.subcore_barrier()

    def edge_loop(tbl):
        def swait(b):
            pltpu.make_async_copy(tbl.at[pl.ds(0, CW)],
                                  acc_s.at[pl.ds(0, CW)], ssems[b]).wait()

        @pl.loop(0, CH2 // IB2)
        def _(r):
            pltpu.sync_copy(src_hbm.at[wid, pl.ds(r * IB2, IB2)], src_v)
            pltpu.sync_copy(dst_hbm.at[wid, pl.ds(r * IB2, IB2)], dst_v)
            for b in range(NB):
                @pl.when(r > 0)
                def _():
                    swait(b)
                pltpu.async_copy(tbl.at[src_v.at[b]], gbufs[b], gsems[b])

            @pl.loop(0, IB2 // NB)
            def _(t):
                for b in range(NB):
                    j = NB * t + b
                    pltpu.make_async_copy(tbl.at[pl.ds(0, CW)], gbufs[b],
                                          gsems[b]).wait()
                    pltpu.async_copy(gbufs[b], acc_s.at[dst_v.at[j]],
                                     ssems[b], add=True)
                    bp = (b + NB - 1) % NB
                    k = j + NB - 1

                    @pl.when((j >= 1) & (k < IB2))
                    def _():
                        swait(bp)
                        pltpu.async_copy(tbl.at[src_v.at[k]], gbufs[bp],
                                         gsems[bp])

        for b in range(NB):
            swait(b)

    @pl.when(c == 0)
    def _():
        edge_loop(hs2a_hbm)

    @pl.when(c == 1)
    def _():
        edge_loop(hs2b_hbm)

    plsc.subcore_barrier()

    @pl.when(c == 0)
    def _():
        pltpu.sync_copy(acc_s.at[pl.ds(base, RPT)], o0_hbm.at[pl.ds(base, RPT)])

    @pl.when(c == 1)
    def _():
        pltpu.sync_copy(acc_s.at[pl.ds(base, RPT)], o1_hbm.at[pl.ds(base, RPT)])


def _dinv(d0_ref, d1_ref):
    deg = d0_ref[:, 0:1] + d1_ref[:, 0:1] + 1.0
    return lax.rsqrt(deg)


def _dense1_body(x_ref, wl_ref, bl_ref, w1_ref, o_ref):
    h0 = jnp.dot(x_ref[...], wl_ref[...], preferred_element_type=f32) + bl_ref[...]
    h0 = jnp.maximum(h0, ALPHA * h0)
    o_ref[...] = jnp.dot(h0, w1_ref[...], preferred_element_type=f32)


def _scale_body(h2_ref, d0_ref, d1_ref, oa_ref, ob_ref):
    hs = h2_ref[...] * _dinv(d0_ref, d1_ref)
    oa_ref[...] = hs[:, :FH]
    ob_ref[...] = hs[:, FH:]


def _dense2_body(ga_ref, gb_ref, ha_ref, hb_ref, d0_ref, d1_ref, b1_ref,
                 w2_ref, oa_ref, ob_ref):
    dinv = _dinv(d0_ref, d1_ref)
    t = jnp.concatenate([ga_ref[...] + ha_ref[...], gb_ref[...] + hb_ref[...]],
                        axis=1)
    t = t * dinv + b1_ref[...]
    h1 = jnp.maximum(t, ALPHA * t)
    hs2 = jnp.dot(h1, w2_ref[...], preferred_element_type=f32) * dinv
    oa_ref[...] = hs2
    ob_ref[...] = hs2


def _final_body(p0_ref, p1_ref, hs2_ref, d0_ref, d1_ref, b2_ref, o_ref):
    z = (p0_ref[...] + p1_ref[...] + hs2_ref[...]) * _dinv(d0_ref, d1_ref)
    z = z + b2_ref[...]
    col = lax.broadcasted_iota(jnp.int32, z.shape, 1)
    valid = col < NCLASS
    zm = jnp.where(valid, z, -jnp.inf)
    m = jnp.max(zm, axis=1, keepdims=True)
    e = jnp.where(valid, jnp.exp(z - m), 0.0)
    lse = jnp.log(jnp.sum(e, axis=1, keepdims=True)) + m
    o_ref[...] = z - lse


def _row_spec(width):
    return pl.BlockSpec((BLK, width), lambda i: (i, 0))


def _full_spec(shape):
    return pl.BlockSpec(shape, lambda i: (0, 0))


_GRID = (NPAD // BLK,)


def _dense1(x_p, Wl, bl2, W1):
    return pl.pallas_call(
        _dense1_body,
        grid=_GRID,
        in_specs=[_row_spec(NFEAT), _full_spec((NFEAT, F1)),
                  _full_spec((1, F1)), _full_spec((F1, F1))],
        out_specs=_row_spec(F1),
        out_shape=jax.ShapeDtypeStruct((NPAD, F1), f32),
    )(x_p, Wl, bl2, W1)


def _scale_split(h2, d0, d1):
    return pl.pallas_call(
        _scale_body,
        grid=_GRID,
        in_specs=[_row_spec(F1), _row_spec(16), _row_spec(16)],
        out_specs=[_row_spec(FH)] * 2,
        out_shape=[jax.ShapeDtypeStruct((NPAD, FH), f32)] * 2,
    )(h2, d0, d1)


def _dense2(ga, gb, ha, hb, d0, d1, b12, W2p):
    return pl.pallas_call(
        _dense2_body,
        grid=_GRID,
        in_specs=[_row_spec(FH), _row_spec(FH), _row_spec(FH), _row_spec(FH),
                  _row_spec(16), _row_spec(16), _full_spec((1, F1)),
                  _full_spec((F1, FP))],
        out_specs=[_row_spec(FP)] * 2,
        out_shape=[jax.ShapeDtypeStruct((NPAD, FP), f32)] * 2,
    )(ga, gb, ha, hb, d0, d1, b12, W2p)


def _final(p0, p1, hs2, d0, d1, b22):
    return pl.pallas_call(
        _final_body,
        grid=_GRID,
        in_specs=[_row_spec(FP), _row_spec(FP), _row_spec(FP),
                  _row_spec(16), _row_spec(16), _full_spec((1, FP))],
        out_specs=_row_spec(FP),
        out_shape=jax.ShapeDtypeStruct((NPAD, FP), f32),
    )(p0, p1, hs2, d0, d1, b22)


def kernel(x, adj, edge_index, Wl, bl, W1, b1, W2, b2):
    del adj
    x_p = jnp.pad(x, ((0, NPAD - N), (0, 0)))
    src = edge_index[0]
    dst = edge_index[1]
    fill1 = jnp.full((E1 - E,), N, jnp.int32)
    fill2 = jnp.full((E2 - E,), N, jnp.int32)
    src1 = jnp.concatenate([src, fill1]).reshape(16, CH1, CW)
    dst1 = jnp.concatenate([dst, fill1]).reshape(16, CH1, CW)
    src2 = jnp.concatenate([src, fill2]).reshape(32, CH2, CW)
    dst2 = jnp.concatenate([dst, fill2]).reshape(32, CH2, CW)
    dstd = jnp.concatenate([dst, fill2]).reshape(32, CHD, 128)
    bl2 = bl.reshape(1, F1)
    b12 = b1.reshape(1, F1)
    b22 = jnp.pad(b2, (0, FP - NCLASS)).reshape(1, FP)
    W2p = jnp.pad(W2, ((0, 0), (0, FP - NCLASS)))

    d0, d1 = _deg_kernel(dstd)
    h2 = _dense1(x_p, Wl, bl2, W1)
    hsa, hsb = _scale_split(h2, d0, d1)
    ga, gb = _agg1_kernel(hsa, hsb, src1, dst1)
    hs2a, hs2b = _dense2(ga, gb, hsa, hsb, d0, d1, b12, W2p)
    p0, p1 = _agg2_kernel(hs2a, hs2b, src2, dst2)
    out = _final(p0, p1, hs2a, d0, d1, b22)
    return out[:N, :NCLASS]
